# baseline (speedup 1.0000x reference)
import functools

import jax
import jax.numpy as jnp
from jax.experimental import pallas as pl

_GAMMA = 0.5


def _vq_body(x_ref, out_ref, c_ref, *, r, n, d):
  for k in range(x_ref.shape[0]):
    xf = x_ref[k]
    x3 = xf.reshape(r, n, d)
    c0 = jnp.sum(x3, axis=1)
    c0 = c0 * jax.lax.rsqrt(jnp.sum(c0 * c0, axis=1, keepdims=True))

    dot = functools.partial(
        jax.lax.dot_general,
        preferred_element_type=jnp.float32,
        precision=jax.lax.Precision.DEFAULT,
    )
    scoresT = dot(c0, xf, dimension_numbers=(((1,), (1,)), ((), ())))
    tokmax = jnp.max(scoresT, axis=0, keepdims=True)
    codemax = jnp.max(scoresT, axis=1, keepdims=True)
    attn = (scoresT == tokmax).astype(jnp.float32) + (
        scoresT == codemax
    ).astype(jnp.float32)
    delta = dot(attn, xf, dimension_numbers=(((1,), (0,)), ((), ())))
    delta = delta * jax.lax.rsqrt(jnp.sum(delta * delta, axis=1, keepdims=True))
    c1 = _GAMMA * c0 + (1.0 - _GAMMA) * delta
    c1 = c1 * jax.lax.rsqrt(jnp.sum(c1 * c1, axis=1, keepdims=True))
    c_ref[k] = c1

    scores1T = dot(c1, xf, dimension_numbers=(((1,), (1,)), ((), ())))
    m1 = jnp.max(scores1T, axis=0, keepdims=True)
    ohT = (scores1T == m1).astype(jnp.float32)
    iota0 = jax.lax.broadcasted_iota(jnp.int32, (d, d), 0)
    iota1 = jax.lax.broadcasted_iota(jnp.int32, (d, d), 1)
    eye = (iota0 == iota1).astype(jnp.float32)
    out_ref[k] = dot(ohT, eye, dimension_numbers=(((0,), (0,)), ((), ())))


def kernel(x):
    b, h, r, n, d = x.shape
    bh = b * h
    l = r * n
    xf = x.reshape(bh, l, d)
    out, c = pl.pallas_call(
        functools.partial(_vq_body, r=r, n=n, d=d),
        grid=(bh // 2,),
        in_specs=[pl.BlockSpec((2, l, d), lambda i: (i, 0, 0))],
        out_specs=[
            pl.BlockSpec((2, l, d), lambda i: (i, 0, 0)),
            pl.BlockSpec((2, r, d), lambda i: (i, 0, 0)),
        ],
        out_shape=[
            jax.ShapeDtypeStruct((bh, l, d), jnp.float32),
            jax.ShapeDtypeStruct((bh, r, d), jnp.float32),
        ],
    )(xf)
    return out.reshape(b, h, r, n, d), c.reshape(b, h, r, d)


# speedup vs baseline: 1.0951x; 1.0951x over previous
import functools

import jax
import jax.numpy as jnp
from jax.experimental import pallas as pl

_GAMMA = 0.5


def _vq_body(x_ref, out_ref, c_ref, *, r, n, d):
    xf = x_ref[0]
    x3 = xf.reshape(r, n, d)
    c0 = jnp.sum(x3, axis=1)
    c0 = c0 * jax.lax.rsqrt(jnp.sum(c0 * c0, axis=1, keepdims=True))

    dot = functools.partial(
        jax.lax.dot_general,
        preferred_element_type=jnp.float32,
        precision=jax.lax.Precision.DEFAULT,
    )
    scoresT = dot(c0, xf, dimension_numbers=(((1,), (1,)), ((), ())))
    tokmax = jnp.max(scoresT, axis=0, keepdims=True)
    codemax = jnp.max(scoresT, axis=1, keepdims=True)
    attn = (scoresT == tokmax).astype(jnp.float32) + (
        scoresT == codemax
    ).astype(jnp.float32)
    delta = dot(attn, xf, dimension_numbers=(((1,), (0,)), ((), ())))
    delta = delta * jax.lax.rsqrt(jnp.sum(delta * delta, axis=1, keepdims=True))
    c1 = _GAMMA * c0 + (1.0 - _GAMMA) * delta
    c1 = c1 * jax.lax.rsqrt(jnp.sum(c1 * c1, axis=1, keepdims=True))
    c_ref[0] = c1

    scores1T = dot(c1, xf, dimension_numbers=(((1,), (1,)), ((), ())))
    m1 = jnp.max(scores1T, axis=0, keepdims=True)
    ohT = (scores1T == m1).astype(jnp.float32)
    out_ref[0] = jnp.swapaxes(ohT, 0, 1)


def kernel(x):
    b, h, r, n, d = x.shape
    bh = b * h
    l = r * n
    xf = x.reshape(bh, l, d)
    out, c = pl.pallas_call(
        functools.partial(_vq_body, r=r, n=n, d=d),
        grid=(bh,),
        in_specs=[pl.BlockSpec((1, l, d), lambda i: (i, 0, 0))],
        out_specs=[
            pl.BlockSpec((1, l, d), lambda i: (i, 0, 0)),
            pl.BlockSpec((1, r, d), lambda i: (i, 0, 0)),
        ],
        out_shape=[
            jax.ShapeDtypeStruct((bh, l, d), jnp.float32),
            jax.ShapeDtypeStruct((bh, r, d), jnp.float32),
        ],
    )(xf)
    return out.reshape(b, h, r, n, d), c.reshape(b, h, r, d)
